# Initial kernel scaffold; baseline (speedup 1.0000x reference)
import jax
import jax.numpy as jnp
from jax.experimental import pallas as pl


def kernel(table):
    raise NotImplementedError("write your pallas kernel here")


# speedup vs baseline: 3.4532x; 3.4532x over previous
import jax
import jax.numpy as jnp
from jax.experimental import pallas as pl

ROWS = 100000
D = 128
BLOCK = 2000


def _body(x_ref, o_ref):
    x = x_ref[...]
    ss = jnp.sum(x * x, axis=-1, keepdims=True)
    o_ref[...] = x * jax.lax.rsqrt(jnp.maximum(ss, 1e-24))


def kernel(table):
    return pl.pallas_call(
        _body,
        grid=(ROWS // BLOCK,),
        in_specs=[pl.BlockSpec((BLOCK, D), lambda i: (i, 0))],
        out_specs=pl.BlockSpec((BLOCK, D), lambda i: (i, 0)),
        out_shape=jax.ShapeDtypeStruct((ROWS, D), jnp.float32),
    )(table)
